# baseline (speedup 1.0000x reference)
import functools

import jax
import jax.numpy as jnp
from jax import lax
from jax.experimental import pallas as pl
from jax.experimental.pallas import tpu as pltpu
from jax.experimental.pallas import tpu_sc as plsc

N = 2048
E = 32768
H = 8
MAX_D = 10
TBL = MAX_D + 2

_NC = 2
_NS = 16
_NT = _NC * _NS


@functools.cache
def _sc_mesh():
    return plsc.VectorSubcoreMesh(
        core_axis_name="c", subcore_axis_name="s",
        num_cores=_NC, num_subcores=_NS)


_WROWS = 32
_PASS = N // (_NT * _WROWS)
_ECHUNK = 8192


def _adj_body(edge_hbm, adj_hbm, win, srcv, dstv):
    c = lax.axis_index("c")
    s = lax.axis_index("s")
    tid = c * _NS + s
    ones16 = jnp.ones((16,), jnp.float32)

    for p in range(_PASS):
        rowbase = tid * (_WROWS * _PASS) + p * _WROWS

        @plsc.parallel_loop(0, _WROWS * N // 16, unroll=4)
        def zro(i):
            win[pl.ds(i * 16, 16)] = jnp.zeros((16,), jnp.float32)

        for ec in range(E // _ECHUNK):
            pltpu.sync_copy(edge_hbm.at[0, pl.ds(ec * _ECHUNK, _ECHUNK)], srcv)
            pltpu.sync_copy(edge_hbm.at[1, pl.ds(ec * _ECHUNK, _ECHUNK)], dstv)

            @plsc.parallel_loop(0, _ECHUNK // 16, unroll=2)
            def grp(t):
                sv = srcv[pl.ds(t * 16, 16)]
                dv = dstv[pl.ds(t * 16, 16)]
                for a, b in ((sv, dv), (dv, sv)):
                    r = a - rowbase
                    ok = (r >= 0) & (r < _WROWS)
                    li = jnp.where(ok, r * N + b, 0)
                    plsc.store_scatter(win, [li], ones16, mask=ok)

        off = pl.multiple_of(rowbase * N, _WROWS * N)
        pltpu.sync_copy(win, adj_hbm.at[pl.ds(off, _WROWS * N)])


@functools.cache
def _adj_scatter():
  return pl.kernel(
    _adj_body,
    out_type=jax.ShapeDtypeStruct((N * N,), jnp.float32),
    mesh=_sc_mesh(),
    compiler_params=pltpu.CompilerParams(use_tc_tiling_on_sc=False, needs_layout_passes=False),
    scratch_types=[
        pltpu.VMEM((_WROWS * N,), jnp.float32),
        pltpu.VMEM((_ECHUNK,), jnp.int32),
        pltpu.VMEM((_ECHUNK,), jnp.int32),
    ],
  )


_RB = 512


def _bfs_body(adj_ref, idxp_ref, adjb_ref):
    i = pl.program_id(0)
    row0 = i * _RB

    @pl.when(i == 0)
    def _():
        adjb_ref[...] = adj_ref[...].reshape(N, N).astype(jnp.int8)

    adjb = adjb_ref[...]
    onesb = jnp.ones((8, _RB), jnp.int8)

    def count(x_s8):
        return jnp.sum(jnp.dot(onesb, x_s8, preferred_element_type=jnp.int32))

    rows = lax.broadcasted_iota(jnp.int32, (_RB, N), 0) + row0
    cols = lax.broadcasted_iota(jnp.int32, (_RB, N), 1)
    reach0 = (rows == cols).astype(jnp.int8)
    reach1 = reach0 | adjb_ref[pl.ds(row0, _RB), :]

    def cond(carry):
        d, done, _, _, _ = carry
        return jnp.logical_and(d <= MAX_D - 1, jnp.logical_not(done))

    def body(carry):
        d, _, prevcnt, reach, s_acc = carry
        nxt = jnp.dot(reach, adjb, preferred_element_type=jnp.int32) > 0
        newr = reach | nxt.astype(jnp.int8)
        cnt = count(newr)
        conv = jnp.logical_or(cnt == prevcnt, cnt == 8 * _RB * N)
        extra = jnp.where(conv, (MAX_D - 1 - d).astype(jnp.float32), 0.0)
        s_acc = s_acc + newr.astype(jnp.float32) * (1.0 + extra)
        return d + 1, conv, cnt, newr, s_acc

    init = (jnp.int32(2), jnp.bool_(False), count(reach1),
            reach1, reach0.astype(jnp.float32) + reach1.astype(jnp.float32))
    _, _, _, _, s_acc = lax.while_loop(cond, body, init)

    idx = (MAX_D + 1.0) - s_acc
    idxp_ref[...] = idx.astype(jnp.int32).reshape(_RB * 16, 128)


@functools.cache
def _bfs():
  return pl.pallas_call(
    _bfs_body,
    grid=(N // _RB,),
    in_specs=[pl.BlockSpec((N * 16, 128), lambda i: (0, 0))],
    out_specs=pl.BlockSpec((_RB * 16, 128), lambda i: (i, 0)),
    out_shape=jax.ShapeDtypeStruct((N * 16, 128), jnp.int32),
    scratch_shapes=[pltpu.VMEM((N, N), jnp.int8)],
  )


_IPT = N // _NT
_ICH = 2
_NCHUNK = _IPT // _ICH
_CIDX = _ICH * N
_CROW = _ICH * 16 * H


def _gather_body(idx_hbm, w_hbm, out_hbm, tblv, ib0, ib1, rb0, rb1,
                 semi0, semi1, semo0, semo1):
    c = lax.axis_index("c")
    s = lax.axis_index("s")
    tid = c * _NS + s
    pbase = tid * (_NCHUNK * _CIDX)
    obase = tid * (_NCHUNK * _CROW)

    pltpu.sync_copy(w_hbm, tblv)

    ibs = (ib0, ib1)
    rbs = (rb0, rb1)
    semis = (semi0, semi1)
    semos = (semo0, semo1)

    def idx_off(g):
        return pl.multiple_of(pbase + g * _CIDX, _CIDX)

    idma = [None, None]
    odma = [None, None]
    idma[0] = pltpu.async_copy(
        idx_hbm.at[pl.ds(idx_off(0), _CIDX)], ibs[0], semis[0])

    for g in range(_NCHUNK):
        cur = g % 2
        nxt = (g + 1) % 2
        if g + 1 < _NCHUNK:
            idma[nxt] = pltpu.async_copy(
                idx_hbm.at[pl.ds(idx_off(g + 1), _CIDX)], ibs[nxt], semis[nxt])
        idma[cur].wait()
        if odma[cur] is not None:
            odma[cur].wait()

        ib = ibs[cur]
        rb = rbs[cur]

        @plsc.parallel_loop(0, _CIDX // 16, unroll=2)
        def block(b):
            iv = ib[pl.ds(b * 16, 16)]
            a0 = iv * H
            row0 = (b >> 7) * (16 * H) + ((b >> 3) & 15) * H
            lane0 = (b & 7) * 16
            for h in range(H):
                v = plsc.load_gather(tblv, [a0 + h])
                rb[row0 + h, pl.ds(lane0, 16)] = v

        odma[cur] = pltpu.async_copy(
            rb, out_hbm.at[pl.ds(pl.multiple_of(obase + g * _CROW, _CROW), _CROW)],
            semos[cur])

    odma[0].wait()
    odma[1].wait()


@functools.cache
def _hgather():
  return pl.kernel(
    _gather_body,
    out_type=jax.ShapeDtypeStruct((N * 16 * H, 128), jnp.float32),
    mesh=_sc_mesh(),
    compiler_params=pltpu.CompilerParams(use_tc_tiling_on_sc=False, needs_layout_passes=False),
    scratch_types=[
        pltpu.VMEM((TBL * H,), jnp.float32),
        pltpu.VMEM((_CIDX,), jnp.int32),
        pltpu.VMEM((_CIDX,), jnp.int32),
        pltpu.VMEM((_CROW, 128), jnp.float32),
        pltpu.VMEM((_CROW, 128), jnp.float32),
        pltpu.SemaphoreType.DMA,
        pltpu.SemaphoreType.DMA,
        pltpu.SemaphoreType.DMA,
        pltpu.SemaphoreType.DMA,
    ],
  )


def kernel(edge_index, num_nodes, spd_bias_weight):
    del num_nodes
    edge_index = edge_index.astype(jnp.int32)
    w = spd_bias_weight.astype(jnp.float32)

    adj = _adj_scatter()(edge_index)
    idx = _bfs()(adj.reshape(N * 16, 128))
    out4 = _hgather()(idx.reshape(-1), w.reshape(-1))
    return out4.reshape(N, 16, H, 128).transpose(0, 1, 3, 2).reshape(N, N, H)
